# Initial kernel scaffold; baseline (speedup 1.0000x reference)
import jax
import jax.numpy as jnp
from jax.experimental import pallas as pl


def kernel(x, w1_fc, b1_fc, w2_fc, b2_fc, w1_cv, b1_cv, w2_cv, b2_cv):
    raise NotImplementedError("write your pallas kernel here")


# speedup vs baseline: 1.0725x; 1.0725x over previous
import functools

import jax
import jax.numpy as jnp
from jax.experimental import pallas as pl
from jax.experimental.pallas import tpu as pltpu


def _fused_kernel(x_ref, w1t_ref, b1_ref, w2t_ref, b2_ref,
                  wfc1_ref, bfc1_ref, wfc2_ref, bfc2_ref,
                  y_ref, x1_ref, x3_ref, xacc_ref, yacc_ref,
                  *, inv_hw, inv_ohw):
    t = pl.program_id(1)
    nt = pl.num_programs(1)

    x = x_ref[0]

    h = jnp.dot(w1t_ref[...], x.astype(jnp.bfloat16),
                preferred_element_type=jnp.float32) + b1_ref[...]
    h = jnp.maximum(h, 0.0)
    y = jnp.dot(w2t_ref[...], h.astype(jnp.bfloat16),
                preferred_element_type=jnp.float32) + b2_ref[...]
    y_ref[0] = y

    xpart = jnp.sum(x, axis=-1).reshape(1, -1)
    ypart = jnp.sum(y).reshape(1, 1)

    @pl.when(t == 0)
    def _():
        xacc_ref[...] = xpart
        yacc_ref[...] = ypart

    @pl.when(t > 0)
    def _():
        xacc_ref[...] += xpart
        yacc_ref[...] += ypart

    @pl.when(t == nt - 1)
    def _():
        pooled = xacc_ref[...] * inv_hw
        hfc = jnp.dot(pooled, wfc1_ref[...],
                      preferred_element_type=jnp.float32) + bfc1_ref[...]
        hfc = jnp.maximum(hfc, 0.0)
        x1 = jnp.dot(hfc, wfc2_ref[...],
                     preferred_element_type=jnp.float32) + bfc2_ref[...]
        x1_ref[0] = x1
        x3_ref[0] = yacc_ref[...] * inv_ohw


def _pick_tile_hw(hw):
    for t in (1024, 512, 256, 128):
        if hw % t == 0:
            return t
    return hw


def kernel(x, w1_fc, b1_fc, w2_fc, b2_fc, w1_cv, b1_cv, w2_cv, b2_cv):
    B, C, H, W = x.shape
    HW = H * W
    hid = w1_cv.shape[1]
    out_dim = w2_cv.shape[1]

    tile_hw = _pick_tile_hw(HW)
    n_tiles = HW // tile_hw

    x_bcl = x.reshape(B, C, HW)
    w1t = w1_cv.T.astype(jnp.bfloat16)
    w2t = w2_cv.T.astype(jnp.bfloat16)
    b1c = b1_cv.reshape(hid, 1)
    b2c = b2_cv.reshape(out_dim, 1)
    bfc1 = b1_fc.reshape(1, hid)
    bfc2 = b2_fc.reshape(1, out_dim)

    body = functools.partial(_fused_kernel,
                             inv_hw=1.0 / HW,
                             inv_ohw=1.0 / (out_dim * HW))

    y, x1o, x3o = pl.pallas_call(
        body,
        grid=(B, n_tiles),
        in_specs=[
            pl.BlockSpec((1, C, tile_hw), lambda b, t: (b, 0, t)),
            pl.BlockSpec((hid, C), lambda b, t: (0, 0)),
            pl.BlockSpec((hid, 1), lambda b, t: (0, 0)),
            pl.BlockSpec((out_dim, hid), lambda b, t: (0, 0)),
            pl.BlockSpec((out_dim, 1), lambda b, t: (0, 0)),
            pl.BlockSpec((C, hid), lambda b, t: (0, 0)),
            pl.BlockSpec((1, hid), lambda b, t: (0, 0)),
            pl.BlockSpec((hid, out_dim), lambda b, t: (0, 0)),
            pl.BlockSpec((1, out_dim), lambda b, t: (0, 0)),
        ],
        out_specs=[
            pl.BlockSpec((1, out_dim, tile_hw), lambda b, t: (b, 0, t)),
            pl.BlockSpec((1, 1, out_dim), lambda b, t: (b, 0, 0)),
            pl.BlockSpec((1, 1, 1), lambda b, t: (b, 0, 0)),
        ],
        out_shape=[
            jax.ShapeDtypeStruct((B, out_dim, HW), jnp.float32),
            jax.ShapeDtypeStruct((B, 1, out_dim), jnp.float32),
            jax.ShapeDtypeStruct((B, 1, 1), jnp.float32),
        ],
        scratch_shapes=[
            pltpu.VMEM((1, C), jnp.float32),
            pltpu.VMEM((1, 1), jnp.float32),
        ],
        compiler_params=pltpu.CompilerParams(
            dimension_semantics=("parallel", "arbitrary")),
    )(x_bcl, w1t, b1c, w2t, b2c, w1_fc, bfc1, w2_fc, bfc2)

    x1 = x1o[:, 0, :]
    x3 = x3o[:, :, 0]
    return x, x1, y, x3
